# baseline (speedup 1.0000x reference)
import functools

import jax
import jax.numpy as jnp
from jax import lax
from jax.experimental import pallas as pl
from jax.experimental.pallas import tpu as pltpu
from jax.experimental.pallas import tpu_sc as plsc

N_CORES = 2
N_SUBCORES = 16
N_WORKERS = N_CORES * N_SUBCORES
LANES = 16
TILE_R = 8
TILE_C = 128

_SC_PARAMS = pltpu.CompilerParams(
    use_tc_tiling_on_sc=True, needs_layout_passes=False)


@functools.cache
def _make_kernels(n_inj: int, n_rows: int, d: int):
    assert n_inj % (N_WORKERS * LANES) == 0
    assert d & (d - 1) == 0
    log2_d = d.bit_length() - 1
    chunk = n_inj // N_WORKERS
    n_batches = n_inj // LANES
    mesh = plsc.VectorSubcoreMesh(core_axis_name="c", subcore_axis_name="s")

    def worker_id():
        return lax.axis_index("s") * N_CORES + lax.axis_index("c")

    def scalar_of(vec16, lane_eq):
        return lax.reduce_max(jnp.where(lane_eq, vec16, 0), axes=(0,))

    def tile_coords(f):
        rb8 = jnp.bitwise_and(jnp.right_shift(f, 3), (d // TILE_R) - 1)
        cb128 = jnp.right_shift(f, log2_d + 7)
        return rb8, cb128, cb128 * (d // TILE_R) + rb8

    def bases_of(idx_v, start, lane_j):
        s = pl.ds(start, LANES)
        lane = lax.iota(jnp.int32, LANES) == lane_j
        rb8, cb128, _ = tile_coords(idx_v[s])
        return (scalar_of(rb8, lane) * TILE_R,
                scalar_of(cb128, lane) * TILE_C)

    @functools.partial(
        pl.kernel,
        mesh=mesh,
        out_type=jax.ShapeDtypeStruct((n_inj * TILE_R, TILE_C),
                                      jnp.float32),
        scratch_types=[
            pltpu.VMEM((n_inj,), jnp.int32),
            pltpu.VMEM((n_inj,), jnp.float32),
            pltpu.VMEM((n_inj,), jnp.float32),
            pltpu.VMEM((chunk * TILE_R, TILE_C), jnp.float32),
            pltpu.SemaphoreType.DMA,
        ],
        compiler_params=_SC_PARAMS,
    )
    def patch(data_hbm, idx_hbm, val_hbm, mask_hbm, tiles_out,
              idx_v, val_v, mask_v, tiles_v, sem):
        wid = worker_id()
        base = wid * chunk
        pltpu.sync_copy(idx_hbm, idx_v)
        pltpu.sync_copy(val_hbm, val_v)
        pltpu.sync_copy(mask_hbm, mask_v)

        copies = []
        for j in range(chunk):
            rb, cb = bases_of(idx_v, base + (j // LANES) * LANES,
                              j % LANES)
            copies.append(pltpu.async_copy(
                data_hbm.at[pl.ds(rb, TILE_R), pl.ds(cb, TILE_C)],
                tiles_v.at[pl.ds(j * TILE_R, TILE_R), :], sem))
        for c in copies:
            c.wait()

        for j in range(chunk):
            s0 = pl.ds(base + (j // LANES) * LANES, LANES)
            lane0 = lax.iota(jnp.int32, LANES) == (j % LANES)
            _, _, tid16 = tile_coords(idx_v[s0])
            t_j = scalar_of(tid16, lane0)

            def scan_body(b, _, j=j, t_j=t_j):
                s = pl.ds(b * LANES, LANES)
                f = idx_v[s]
                _, _, tid = tile_coords(f)
                m = tid == t_j
                hits = lax.reduce_max(jnp.where(m, 1, 0), axes=(0,))

                @pl.when(hits > 0)
                def _():
                    row = jnp.bitwise_and(f, TILE_R - 1) + j * TILE_R
                    col = jnp.bitwise_and(jnp.right_shift(f, log2_d),
                                          TILE_C - 1)
                    cur = plsc.load_gather(tiles_v, [row, col])
                    plsc.store_scatter(tiles_v, [row, col],
                                       val_v[s] + cur * mask_v[s],
                                       mask=m)

                return None

            lax.fori_loop(0, n_batches, scan_body, None)

        pltpu.sync_copy(
            tiles_v, tiles_out.at[pl.ds(base * TILE_R, chunk * TILE_R), :])

    @functools.partial(
        pl.kernel,
        mesh=mesh,
        out_type=(),
        scratch_types=[
            pltpu.VMEM((chunk,), jnp.int32),
            pltpu.VMEM((chunk * TILE_R, TILE_C), jnp.float32),
            pltpu.SemaphoreType.DMA,
        ],
        compiler_params=_SC_PARAMS,
    )
    def writeback(data_ref, idx_hbm, tiles_hbm, idx_v, tiles_v, sem):
        wid = worker_id()
        base = wid * chunk
        pltpu.sync_copy(idx_hbm.at[pl.ds(base, chunk)], idx_v)
        pltpu.sync_copy(
            tiles_hbm.at[pl.ds(base * TILE_R, chunk * TILE_R), :], tiles_v)
        copies = []
        for j in range(chunk):
            rb, cb = bases_of(idx_v, (j // LANES) * LANES, j % LANES)
            copies.append(pltpu.async_copy(
                tiles_v.at[pl.ds(j * TILE_R, TILE_R), :],
                data_ref.at[pl.ds(rb, TILE_R), pl.ds(cb, TILE_C)], sem))
        for c in copies:
            c.wait()

    return patch, writeback


def kernel(x, inj_idx, inj_val, inj_is_add):
    idx = inj_idx.astype(jnp.int32)
    mask = inj_is_add.astype(jnp.float32)
    patch, writeback = _make_kernels(idx.shape[0], *x.shape)
    tiles = patch(x.T, idx, inj_val, mask)
    data = jax.new_ref(x.T)
    writeback(data, idx, tiles)
    return data[...].T


# speedup vs baseline: 1.1520x; 1.1520x over previous
import functools

import jax
import jax.numpy as jnp
from jax import lax
from jax.experimental import pallas as pl
from jax.experimental.pallas import tpu as pltpu
from jax.experimental.pallas import tpu_sc as plsc

N_CORES = 2
N_SUBCORES = 16
N_WORKERS = N_CORES * N_SUBCORES
LANES = 16
TILE_R = 8
TILE_C = 128

_SC_PARAMS = pltpu.CompilerParams(
    use_tc_tiling_on_sc=True, needs_layout_passes=False)


@functools.cache
def _make_kernels(n_inj: int, n_rows: int, d: int):
    assert n_inj % (N_WORKERS * LANES) == 0
    assert d & (d - 1) == 0
    log2_d = d.bit_length() - 1
    chunk = n_inj // N_WORKERS
    n_batches = n_inj // LANES
    mesh = plsc.VectorSubcoreMesh(core_axis_name="c", subcore_axis_name="s")

    def worker_id():
        return lax.axis_index("s") * N_CORES + lax.axis_index("c")

    def scalar_of(vec16, lane_eq):
        return lax.reduce_max(jnp.where(lane_eq, vec16, 0), axes=(0,))

    def tile_coords(f):
        rb8 = jnp.bitwise_and(jnp.right_shift(f, 3), (d // TILE_R) - 1)
        cb128 = jnp.right_shift(f, log2_d + 7)
        return rb8, cb128, cb128 * (d // TILE_R) + rb8

    def bases_of(idx_v, start, lane_j):
        s = pl.ds(start, LANES)
        lane = lax.iota(jnp.int32, LANES) == lane_j
        rb8, cb128, _ = tile_coords(idx_v[s])
        return (scalar_of(rb8, lane) * TILE_R,
                scalar_of(cb128, lane) * TILE_C)

    @functools.partial(
        pl.kernel,
        mesh=mesh,
        out_type=jax.ShapeDtypeStruct((n_inj * TILE_R, TILE_C),
                                      jnp.float32),
        scratch_types=[
            pltpu.VMEM((n_inj,), jnp.int32),
            pltpu.VMEM((n_inj,), jnp.float32),
            pltpu.VMEM((n_inj,), jnp.float32),
            pltpu.VMEM((n_inj,), jnp.int32),
            pltpu.VMEM((n_inj,), jnp.int32),
            pltpu.VMEM((n_inj,), jnp.int32),
            pltpu.VMEM((chunk * TILE_R, TILE_C), jnp.float32),
            pltpu.SemaphoreType.DMA,
        ],
        compiler_params=_SC_PARAMS,
    )
    def patch(data_hbm, idx_hbm, val_hbm, mask_hbm, tiles_out,
              idx_v, val_v, mask_v, tid_v, row_v, col_v, tiles_v, sem):
        wid = worker_id()
        base = wid * chunk
        pltpu.sync_copy(idx_hbm, idx_v)
        pltpu.sync_copy(val_hbm, val_v)
        pltpu.sync_copy(mask_hbm, mask_v)
        for b in range(n_batches):
            s = pl.ds(b * LANES, LANES)
            f = idx_v[s]
            _, _, tid = tile_coords(f)
            tid_v[s] = tid
            row_v[s] = jnp.bitwise_and(f, TILE_R - 1)
            col_v[s] = jnp.bitwise_and(jnp.right_shift(f, log2_d),
                                       TILE_C - 1)

        copies = []
        for j in range(chunk):
            rb, cb = bases_of(idx_v, base + (j // LANES) * LANES,
                              j % LANES)
            copies.append(pltpu.async_copy(
                data_hbm.at[pl.ds(rb, TILE_R), pl.ds(cb, TILE_C)],
                tiles_v.at[pl.ds(j * TILE_R, TILE_R), :], sem))
        for c in copies:
            c.wait()

        for j in range(chunk):
            s0 = pl.ds(base + (j // LANES) * LANES, LANES)
            lane0 = lax.iota(jnp.int32, LANES) == (j % LANES)
            t_j = scalar_of(tid_v[s0], lane0)

            def scan_body(b, _, j=j, t_j=t_j):
                s = pl.ds(b * LANES, LANES)
                m = tid_v[s] == t_j
                row = row_v[s] + j * TILE_R
                col = col_v[s]
                cur = plsc.load_gather(tiles_v, [row, col])
                plsc.store_scatter(tiles_v, [row, col],
                                   val_v[s] + cur * mask_v[s], mask=m)
                return None

            lax.fori_loop(0, n_batches, scan_body, None)

        pltpu.sync_copy(
            tiles_v, tiles_out.at[pl.ds(base * TILE_R, chunk * TILE_R), :])

    @functools.partial(
        pl.kernel,
        mesh=mesh,
        out_type=(),
        scratch_types=[
            pltpu.VMEM((chunk,), jnp.int32),
            pltpu.VMEM((chunk * TILE_R, TILE_C), jnp.float32),
            pltpu.SemaphoreType.DMA,
        ],
        compiler_params=_SC_PARAMS,
    )
    def writeback(data_ref, idx_hbm, tiles_hbm, idx_v, tiles_v, sem):
        wid = worker_id()
        base = wid * chunk
        pltpu.sync_copy(idx_hbm.at[pl.ds(base, chunk)], idx_v)
        pltpu.sync_copy(
            tiles_hbm.at[pl.ds(base * TILE_R, chunk * TILE_R), :], tiles_v)
        copies = []
        for j in range(chunk):
            rb, cb = bases_of(idx_v, (j // LANES) * LANES, j % LANES)
            copies.append(pltpu.async_copy(
                tiles_v.at[pl.ds(j * TILE_R, TILE_R), :],
                data_ref.at[pl.ds(rb, TILE_R), pl.ds(cb, TILE_C)], sem))
        for c in copies:
            c.wait()

    return patch, writeback


def kernel(x, inj_idx, inj_val, inj_is_add):
    idx = inj_idx.astype(jnp.int32)
    mask = inj_is_add.astype(jnp.float32)
    patch, writeback = _make_kernels(idx.shape[0], *x.shape)
    tiles = patch(x.T, idx, inj_val, mask)
    data = jax.new_ref(x.T)
    writeback(data, idx, tiles)
    return data[...].T


# speedup vs baseline: 1.1923x; 1.0349x over previous
import functools

import jax
import jax.numpy as jnp
from jax import lax
from jax.experimental import pallas as pl
from jax.experimental.pallas import tpu as pltpu
from jax.experimental.pallas import tpu_sc as plsc

N_CORES = 2
N_SUBCORES = 16
N_WORKERS = N_CORES * N_SUBCORES
LANES = 16
TILE_R = 8
TILE_C = 128

_SC_PARAMS = pltpu.CompilerParams(
    use_tc_tiling_on_sc=True, needs_layout_passes=False)


@functools.cache
def _make_inject(n_inj: int, n_rows: int, d: int):
    assert n_inj % (N_WORKERS * LANES) == 0
    assert d & (d - 1) == 0
    log2_d = d.bit_length() - 1
    chunk = n_inj // N_WORKERS
    n_batches = n_inj // LANES
    mesh = plsc.VectorSubcoreMesh(core_axis_name="c", subcore_axis_name="s")

    def scalar_of(vec16, lane_eq):
        return lax.reduce_max(jnp.where(lane_eq, vec16, 0), axes=(0,))

    def tile_coords(f):
        rb8 = jnp.bitwise_and(jnp.right_shift(f, 3), (d // TILE_R) - 1)
        cb128 = jnp.right_shift(f, log2_d + 7)
        return rb8, cb128, cb128 * (d // TILE_R) + rb8

    def bases_of(idx_v, start, lane_j):
        s = pl.ds(start, LANES)
        lane = lax.iota(jnp.int32, LANES) == lane_j
        rb8, cb128, _ = tile_coords(idx_v[s])
        return (scalar_of(rb8, lane) * TILE_R,
                scalar_of(cb128, lane) * TILE_C)

    @functools.partial(
        pl.kernel,
        mesh=mesh,
        out_type=(),
        scratch_types=[
            pltpu.VMEM((n_inj,), jnp.int32),
            pltpu.VMEM((n_inj,), jnp.float32),
            pltpu.VMEM((n_inj,), jnp.float32),
            pltpu.VMEM((n_inj,), jnp.int32),
            pltpu.VMEM((n_inj,), jnp.int32),
            pltpu.VMEM((n_inj,), jnp.int32),
            pltpu.VMEM((chunk * TILE_R, TILE_C), jnp.float32),
            pltpu.SemaphoreType.DMA,
        ],
        compiler_params=_SC_PARAMS,
    )
    def inject(data_ref, x_hbm, idx_hbm, val_hbm, mask_hbm,
               idx_v, val_v, mask_v, tid_v, row_v, col_v, tiles_v, sem):
        wid = lax.axis_index("s") * N_CORES + lax.axis_index("c")
        base = wid * chunk
        pltpu.sync_copy(idx_hbm, idx_v)
        pltpu.sync_copy(val_hbm, val_v)
        pltpu.sync_copy(mask_hbm, mask_v)
        for b in range(n_batches):
            s = pl.ds(b * LANES, LANES)
            f = idx_v[s]
            _, _, tid = tile_coords(f)
            tid_v[s] = tid
            row_v[s] = jnp.bitwise_and(f, TILE_R - 1)
            col_v[s] = jnp.bitwise_and(jnp.right_shift(f, log2_d),
                                       TILE_C - 1)

        copies = []
        for j in range(chunk):
            rb, cb = bases_of(idx_v, base + (j // LANES) * LANES,
                              j % LANES)
            copies.append(pltpu.async_copy(
                x_hbm.at[pl.ds(rb, TILE_R), pl.ds(cb, TILE_C)],
                tiles_v.at[pl.ds(j * TILE_R, TILE_R), :], sem))
        for c in copies:
            c.wait()

        for j in range(chunk):
            s0 = pl.ds(base + (j // LANES) * LANES, LANES)
            lane0 = lax.iota(jnp.int32, LANES) == (j % LANES)
            t_j = scalar_of(tid_v[s0], lane0)

            def scan_body(b, _, j=j, t_j=t_j):
                s = pl.ds(b * LANES, LANES)
                m = tid_v[s] == t_j
                row = row_v[s] + j * TILE_R
                col = col_v[s]
                cur = plsc.load_gather(tiles_v, [row, col])
                plsc.store_scatter(tiles_v, [row, col],
                                   val_v[s] + cur * mask_v[s], mask=m)
                return None

            lax.fori_loop(0, n_batches, scan_body, None)

        copies = []
        for j in range(chunk):
            rb, cb = bases_of(idx_v, base + (j // LANES) * LANES,
                              j % LANES)
            copies.append(pltpu.async_copy(
                tiles_v.at[pl.ds(j * TILE_R, TILE_R), :],
                data_ref.at[pl.ds(rb, TILE_R), pl.ds(cb, TILE_C)], sem))
        for c in copies:
            c.wait()

    return inject


def kernel(x, inj_idx, inj_val, inj_is_add):
    idx = inj_idx.astype(jnp.int32)
    mask = inj_is_add.astype(jnp.float32)
    data = jax.new_ref(x.T)
    _make_inject(idx.shape[0], *x.shape)(data, x.T, idx, inj_val, mask)
    return data[...].T


# speedup vs baseline: 1.3123x; 1.1007x over previous
import functools

import jax
import jax.numpy as jnp
from jax import lax
from jax.experimental import pallas as pl
from jax.experimental.pallas import tpu as pltpu
from jax.experimental.pallas import tpu_sc as plsc

N_CORES = 2
N_SUBCORES = 16
N_WORKERS = N_CORES * N_SUBCORES
LANES = 16
TILE_R = 8
TILE_C = 128

_SC_PARAMS = pltpu.CompilerParams(
    use_tc_tiling_on_sc=True, needs_layout_passes=False)


@functools.cache
def _make_inject(n_inj: int, n_rows: int, d: int):
    assert n_inj % (N_WORKERS * LANES) == 0
    assert d & (d - 1) == 0
    log2_d = d.bit_length() - 1
    chunk = n_inj // N_WORKERS
    n_batches = n_inj // LANES
    mesh = plsc.VectorSubcoreMesh(core_axis_name="c", subcore_axis_name="s")

    def scalar_of(vec16, lane_eq):
        return lax.reduce_max(jnp.where(lane_eq, vec16, 0), axes=(0,))

    def tile_coords(f):
        rb8 = jnp.bitwise_and(jnp.right_shift(f, 3), (d // TILE_R) - 1)
        cb128 = jnp.right_shift(f, log2_d + 7)
        return rb8, cb128, cb128 * (d // TILE_R) + rb8

    def bases_of(idx_v, start, lane_j):
        s = pl.ds(start, LANES)
        lane = lax.iota(jnp.int32, LANES) == lane_j
        rb8, cb128, _ = tile_coords(idx_v[s])
        return (scalar_of(rb8, lane) * TILE_R,
                scalar_of(cb128, lane) * TILE_C)

    @functools.partial(
        pl.kernel,
        mesh=mesh,
        out_type=(),
        scratch_types=[
            pltpu.VMEM((n_inj,), jnp.int32),
            pltpu.VMEM((n_inj,), jnp.float32),
            pltpu.VMEM((n_inj,), jnp.float32),
            pltpu.VMEM((n_inj,), jnp.int32),
            pltpu.VMEM((n_inj,), jnp.int32),
            pltpu.VMEM((n_inj,), jnp.int32),
            pltpu.VMEM((chunk * TILE_R, TILE_C), jnp.float32),
            pltpu.SemaphoreType.DMA,
        ],
        compiler_params=_SC_PARAMS,
    )
    def inject(data_ref, x_hbm, idx_hbm, val_hbm, mask_hbm,
               idx_v, val_v, mask_v, tid_v, row_v, col_v, tiles_v, sem):
        wid = lax.axis_index("s") * N_CORES + lax.axis_index("c")
        base = wid * chunk
        pltpu.sync_copy(idx_hbm, idx_v)
        pltpu.sync_copy(val_hbm, val_v)
        pltpu.sync_copy(mask_hbm, mask_v)
        for b in range(n_batches):
            s = pl.ds(b * LANES, LANES)
            f = idx_v[s]
            _, _, tid = tile_coords(f)
            tid_v[s] = tid
            row_v[s] = jnp.bitwise_and(f, TILE_R - 1)
            col_v[s] = jnp.bitwise_and(jnp.right_shift(f, log2_d),
                                       TILE_C - 1)

        copies = []
        for j in range(chunk):
            rb, cb = bases_of(idx_v, base + (j // LANES) * LANES,
                              j % LANES)
            copies.append(pltpu.async_copy(
                x_hbm.at[pl.ds(rb, TILE_R), pl.ds(cb, TILE_C)],
                tiles_v.at[pl.ds(j * TILE_R, TILE_R), :], sem))
        for c in copies:
            c.wait()

        t_js = []
        for j in range(chunk):
            s0 = pl.ds(base + (j // LANES) * LANES, LANES)
            lane0 = lax.iota(jnp.int32, LANES) == (j % LANES)
            t_js.append(scalar_of(tid_v[s0], lane0))

        def scan_body(b, _):
            s = pl.ds(b * LANES, LANES)
            tid = tid_v[s]
            row0 = row_v[s]
            col = col_v[s]
            val = val_v[s]
            msk = mask_v[s]
            for j in range(chunk):
                m = tid == t_js[j]
                row = row0 + j * TILE_R
                cur = plsc.load_gather(tiles_v, [row, col])
                plsc.store_scatter(tiles_v, [row, col],
                                   val + cur * msk, mask=m)
            return None

        lax.fori_loop(0, n_batches, scan_body, None)

        copies = []
        for j in range(chunk):
            rb, cb = bases_of(idx_v, base + (j // LANES) * LANES,
                              j % LANES)
            copies.append(pltpu.async_copy(
                tiles_v.at[pl.ds(j * TILE_R, TILE_R), :],
                data_ref.at[pl.ds(rb, TILE_R), pl.ds(cb, TILE_C)], sem))
        for c in copies:
            c.wait()

    return inject


def kernel(x, inj_idx, inj_val, inj_is_add):
    idx = inj_idx.astype(jnp.int32)
    mask = inj_is_add.astype(jnp.float32)
    data = jax.new_ref(x.T)
    _make_inject(idx.shape[0], *x.shape)(data, x.T, idx, inj_val, mask)
    return data[...].T


# speedup vs baseline: 1.3179x; 1.0043x over previous
import functools

import jax
import jax.numpy as jnp
from jax import lax
from jax.experimental import pallas as pl
from jax.experimental.pallas import tpu as pltpu
from jax.experimental.pallas import tpu_sc as plsc

N_CORES = 2
N_SUBCORES = 16
N_WORKERS = N_CORES * N_SUBCORES
LANES = 16
TILE_R = 8
TILE_C = 128

_SC_PARAMS = pltpu.CompilerParams(
    use_tc_tiling_on_sc=True, needs_layout_passes=False)


@functools.cache
def _make_inject(n_inj: int, n_rows: int, d: int):
    assert n_inj % (N_WORKERS * LANES) == 0
    assert d & (d - 1) == 0
    log2_d = d.bit_length() - 1
    chunk = n_inj // N_WORKERS
    n_batches = n_inj // LANES
    mesh = plsc.VectorSubcoreMesh(core_axis_name="c", subcore_axis_name="s")

    def scalar_of(vec16, lane_eq):
        return lax.reduce_max(jnp.where(lane_eq, vec16, 0), axes=(0,))

    def tile_coords(f):
        rb8 = jnp.bitwise_and(jnp.right_shift(f, 3), (d // TILE_R) - 1)
        cb128 = jnp.right_shift(f, log2_d + 7)
        return rb8, cb128, cb128 * (d // TILE_R) + rb8

    def bases_of(idx_v, start, lane_j):
        s = pl.ds(start, LANES)
        lane = lax.iota(jnp.int32, LANES) == lane_j
        rb8, cb128, _ = tile_coords(idx_v[s])
        return (scalar_of(rb8, lane) * TILE_R,
                scalar_of(cb128, lane) * TILE_C)

    @functools.partial(
        pl.kernel,
        mesh=mesh,
        out_type=(),
        scratch_types=[
            pltpu.VMEM((n_inj,), jnp.int32),
            pltpu.VMEM((n_inj,), jnp.float32),
            pltpu.VMEM((n_inj,), jnp.float32),
            pltpu.VMEM((n_inj,), jnp.int32),
            pltpu.VMEM((n_inj,), jnp.int32),
            pltpu.VMEM((n_inj,), jnp.int32),
            pltpu.VMEM((chunk * TILE_R, TILE_C), jnp.float32),
            pltpu.SemaphoreType.DMA,
        ],
        compiler_params=_SC_PARAMS,
    )
    def inject(data_ref, x_hbm, idx_hbm, val_hbm, mask_hbm,
               idx_v, val_v, mask_v, tid_v, row_v, col_v, tiles_v, sem):
        wid = lax.axis_index("s") * N_CORES + lax.axis_index("c")
        base = wid * chunk
        pltpu.sync_copy(idx_hbm, idx_v)

        tile_bases = []
        copies = []
        for j in range(chunk):
            rb, cb = bases_of(idx_v, base + (j // LANES) * LANES,
                              j % LANES)
            tile_bases.append((rb, cb))
            copies.append(pltpu.async_copy(
                x_hbm.at[pl.ds(rb, TILE_R), pl.ds(cb, TILE_C)],
                tiles_v.at[pl.ds(j * TILE_R, TILE_R), :], sem))

        pltpu.sync_copy(val_hbm, val_v)
        pltpu.sync_copy(mask_hbm, mask_v)
        for b in range(n_batches):
            s = pl.ds(b * LANES, LANES)
            f = idx_v[s]
            _, _, tid = tile_coords(f)
            tid_v[s] = tid
            row_v[s] = jnp.bitwise_and(f, TILE_R - 1)
            col_v[s] = jnp.bitwise_and(jnp.right_shift(f, log2_d),
                                       TILE_C - 1)
        for c in copies:
            c.wait()

        t_js = []
        for j in range(chunk):
            s0 = pl.ds(base + (j // LANES) * LANES, LANES)
            lane0 = lax.iota(jnp.int32, LANES) == (j % LANES)
            t_js.append(scalar_of(tid_v[s0], lane0))

        def scan_body(b, _):
            s = pl.ds(b * LANES, LANES)
            tid = tid_v[s]
            row0 = row_v[s]
            col = col_v[s]
            val = val_v[s]
            msk = mask_v[s]
            for j in range(chunk):
                m = tid == t_js[j]
                row = row0 + j * TILE_R
                cur = plsc.load_gather(tiles_v, [row, col])
                plsc.store_scatter(tiles_v, [row, col],
                                   val + cur * msk, mask=m)
            return None

        lax.fori_loop(0, n_batches, scan_body, None)

        copies = []
        for j in range(chunk):
            rb, cb = tile_bases[j]
            copies.append(pltpu.async_copy(
                tiles_v.at[pl.ds(j * TILE_R, TILE_R), :],
                data_ref.at[pl.ds(rb, TILE_R), pl.ds(cb, TILE_C)], sem))
        for c in copies:
            c.wait()

    return inject


def kernel(x, inj_idx, inj_val, inj_is_add):
    idx = inj_idx.astype(jnp.int32)
    mask = inj_is_add.astype(jnp.float32)
    data = jax.new_ref(x.T)
    _make_inject(idx.shape[0], *x.shape)(data, x.T, idx, inj_val, mask)
    return data[...].T
